# baseline (speedup 1.0000x reference)
import jax
import jax.numpy as jnp
from jax.experimental import pallas as pl
from jax.experimental.pallas import tpu as pltpu

N = 10000
A = 32
NODE_DIM = 256
ATTR_DIM = 512
HIDDEN = 128

BLOCK = 10000
NB = N // BLOCK

_BF = jnp.bfloat16
_F32 = jnp.float32


def _fused_kernel(v_ref, qa_ref, emb_ref, W_in_ref, b_in_ref, Wa0_ref,
                  Ua0_ref, ba0_ref, Wv0_ref, Wv1_ref, bv_ref, Uv0_ref,
                  Uv1_ref, W_out_ref, b_out_ref, out_ref,
                  h1_s, sum_s, c_s, ha0_s):
    i = pl.program_id(0)

    @pl.when(i == 0)
    def _():
        sum_s[...] = jnp.zeros_like(sum_s)
        col = jax.lax.broadcasted_iota(jnp.int32, (A, ATTR_DIM), 1)
        onehot = (col == qa_ref[...]).astype(_F32)
        h_a0 = jnp.dot(onehot, emb_ref[...], preferred_element_type=_F32)
        ha0_s[...] = h_a0
        mean_a0 = jnp.sum(h_a0, axis=0, keepdims=True) * (1.0 / A)
        c_s[0:1, :] = (jnp.dot(mean_a0, Wv0_ref[...],
                               preferred_element_type=_F32) + bv_ref[0:1, :])

    @pl.when(i < NB)
    def _():
        vb = v_ref[...]
        sum_s[...] += jnp.sum(vb, axis=0, keepdims=True)
        h0 = (jnp.dot(vb.astype(_BF), W_in_ref[...].astype(_BF),
                      preferred_element_type=_F32) + b_in_ref[...])
        h1 = jax.nn.relu(
            jnp.dot(h0.astype(_BF), Uv0_ref[...].astype(_BF),
                    preferred_element_type=_F32) + c_s[0:1, :])
        h1_s[pl.ds(i * BLOCK, BLOCK), :] = h1.astype(_BF)

    @pl.when(i == NB)
    def _():
        mean_v0 = (jnp.dot(sum_s[...] * (1.0 / N), W_in_ref[...],
                           preferred_element_type=_F32) + b_in_ref[...])
        h_a1 = jax.nn.relu(
            jnp.dot(mean_v0, Wa0_ref[...], preferred_element_type=_F32)
            + jnp.dot(ha0_s[...], Ua0_ref[...], preferred_element_type=_F32)
            + ba0_ref[...])
        mean_a1 = jnp.sum(h_a1, axis=0, keepdims=True) * (1.0 / A)
        c_s[1:2, :] = (jnp.dot(mean_a1, Wv1_ref[...],
                               preferred_element_type=_F32) + bv_ref[1:2, :])

    @pl.when(i >= NB)
    def _():
        j = i - NB
        h1b = h1_s[pl.ds(j * BLOCK, BLOCK), :]
        h2 = jax.nn.relu(
            jnp.dot(h1b, Uv1_ref[...].astype(_BF),
                    preferred_element_type=_F32) + c_s[1:2, :])
        out_ref[...] = (
            jnp.dot(h2.astype(_BF), W_out_ref[...].astype(_BF),
                    preferred_element_type=_F32) + b_out_ref[...])


def _full(shape):
    nd = len(shape)
    return pl.BlockSpec(shape, lambda i: (0,) * nd)


@jax.jit
def kernel(v, query_attrs, emb_table, W_in, b_in, Wa, Ua, ba, Wv, Uv, bv,
           W_out, b_out):
    qa = query_attrs.astype(jnp.int32).reshape(A, 1)
    b_in2 = b_in.reshape(1, HIDDEN)
    ba0 = ba[0].reshape(1, HIDDEN)
    b_out2 = b_out.reshape(1, NODE_DIM)

    out = pl.pallas_call(
        _fused_kernel,
        grid=(2 * NB,),
        in_specs=[
            pl.BlockSpec((BLOCK, NODE_DIM),
                         lambda i: (jnp.minimum(i, NB - 1), 0)),
            _full((A, 1)),
            _full((ATTR_DIM, HIDDEN)),
            _full((NODE_DIM, HIDDEN)),
            _full((1, HIDDEN)),
            _full((HIDDEN, HIDDEN)),
            _full((HIDDEN, HIDDEN)),
            _full((1, HIDDEN)),
            _full((HIDDEN, HIDDEN)),
            _full((HIDDEN, HIDDEN)),
            _full((2, HIDDEN)),
            _full((HIDDEN, HIDDEN)),
            _full((HIDDEN, HIDDEN)),
            _full((HIDDEN, NODE_DIM)),
            _full((1, NODE_DIM)),
        ],
        out_specs=pl.BlockSpec((BLOCK, NODE_DIM),
                               lambda i: (jnp.maximum(i - NB, 0), 0)),
        out_shape=jax.ShapeDtypeStruct((N, NODE_DIM), jnp.float32),
        scratch_shapes=[
            pltpu.VMEM((N, HIDDEN), _BF),
            pltpu.VMEM((1, NODE_DIM), _F32),
            pltpu.VMEM((2, HIDDEN), _F32),
            pltpu.VMEM((A, HIDDEN), _F32),
        ],
    )(v, qa, emb_table, W_in, b_in2, Wa[0], Ua[0], ba0, Wv[0], Wv[1], bv,
      Uv[0], Uv[1], W_out, b_out2)

    return out


# speedup vs baseline: 1.1141x; 1.1141x over previous
import jax
import jax.numpy as jnp
from jax.experimental import pallas as pl
from jax.experimental.pallas import tpu as pltpu

N = 10000
A = 32
NODE_DIM = 256
ATTR_DIM = 512
HIDDEN = 128

BLOCK = 5000
NB = N // BLOCK

_BF = jnp.bfloat16
_F32 = jnp.float32


def _fused_kernel(v_ref, qa_ref, emb_ref, W_in_ref, b_in_ref, Wa0_ref,
                  Ua0_ref, ba0_ref, Wv0_ref, Wv1_ref, bv_ref, Uv0_ref,
                  Uv1_ref, W_out_ref, b_out_ref, out_ref,
                  h1_s, sum_s, c_s, ha0_s):
    i = pl.program_id(0)

    @pl.when(i == 0)
    def _():
        sum_s[...] = jnp.zeros_like(sum_s)
        col = jax.lax.broadcasted_iota(jnp.int32, (A, ATTR_DIM), 1)
        onehot = (col == qa_ref[...]).astype(_F32)
        h_a0 = jnp.dot(onehot, emb_ref[...], preferred_element_type=_F32)
        ha0_s[...] = h_a0
        mean_a0 = jnp.sum(h_a0, axis=0, keepdims=True) * (1.0 / A)
        c_s[0:1, :] = (jnp.dot(mean_a0, Wv0_ref[...],
                               preferred_element_type=_F32) + bv_ref[0:1, :])

    @pl.when(i < NB)
    def _():
        vb = v_ref[...]
        h0 = (jnp.dot(vb.astype(_BF), W_in_ref[...].astype(_BF),
                      preferred_element_type=_F32) + b_in_ref[...])
        sum_s[...] += jnp.sum(h0, axis=0, keepdims=True)
        h1 = jax.nn.relu(
            jnp.dot(h0.astype(_BF), Uv0_ref[...].astype(_BF),
                    preferred_element_type=_F32) + c_s[0:1, :])
        h1_s[pl.ds(i * BLOCK, BLOCK), :] = h1.astype(_BF)

    @pl.when(i == NB)
    def _():
        mean_v0 = sum_s[...] * (1.0 / N)
        h_a1 = jax.nn.relu(
            jnp.dot(mean_v0, Wa0_ref[...], preferred_element_type=_F32)
            + jnp.dot(ha0_s[...], Ua0_ref[...], preferred_element_type=_F32)
            + ba0_ref[...])
        mean_a1 = jnp.sum(h_a1, axis=0, keepdims=True) * (1.0 / A)
        c_s[1:2, :] = (jnp.dot(mean_a1, Wv1_ref[...],
                               preferred_element_type=_F32) + bv_ref[1:2, :])

    @pl.when(i >= NB)
    def _():
        j = i - NB
        h1b = h1_s[pl.ds(j * BLOCK, BLOCK), :]
        h2 = jax.nn.relu(
            jnp.dot(h1b, Uv1_ref[...].astype(_BF),
                    preferred_element_type=_F32) + c_s[1:2, :])
        out_ref[...] = (
            jnp.dot(h2.astype(_BF), W_out_ref[...].astype(_BF),
                    preferred_element_type=_F32) + b_out_ref[...])


def _full(shape):
    nd = len(shape)
    return pl.BlockSpec(shape, lambda i: (0,) * nd)


@jax.jit
def kernel(v, query_attrs, emb_table, W_in, b_in, Wa, Ua, ba, Wv, Uv, bv,
           W_out, b_out):
    qa = query_attrs.astype(jnp.int32).reshape(A, 1)
    b_in2 = b_in.reshape(1, HIDDEN)
    ba0 = ba[0].reshape(1, HIDDEN)
    b_out2 = b_out.reshape(1, NODE_DIM)

    out = pl.pallas_call(
        _fused_kernel,
        grid=(2 * NB,),
        in_specs=[
            pl.BlockSpec((BLOCK, NODE_DIM),
                         lambda i: (jnp.minimum(i, NB - 1), 0)),
            _full((A, 1)),
            _full((ATTR_DIM, HIDDEN)),
            _full((NODE_DIM, HIDDEN)),
            _full((1, HIDDEN)),
            _full((HIDDEN, HIDDEN)),
            _full((HIDDEN, HIDDEN)),
            _full((1, HIDDEN)),
            _full((HIDDEN, HIDDEN)),
            _full((HIDDEN, HIDDEN)),
            _full((2, HIDDEN)),
            _full((HIDDEN, HIDDEN)),
            _full((HIDDEN, HIDDEN)),
            _full((HIDDEN, NODE_DIM)),
            _full((1, NODE_DIM)),
        ],
        out_specs=pl.BlockSpec((BLOCK, NODE_DIM),
                               lambda i: (jnp.maximum(i - NB, 0), 0)),
        out_shape=jax.ShapeDtypeStruct((N, NODE_DIM), jnp.float32),
        scratch_shapes=[
            pltpu.VMEM((N, HIDDEN), _BF),
            pltpu.VMEM((1, HIDDEN), _F32),
            pltpu.VMEM((2, HIDDEN), _F32),
            pltpu.VMEM((A, HIDDEN), _F32),
        ],
    )(v, qa, emb_table, W_in, b_in2, Wa[0], Ua[0], ba0, Wv[0], Wv[1], bv,
      Uv[0], Uv[1], W_out, b_out2)

    return out
